# Initial kernel scaffold; baseline (speedup 1.0000x reference)
import jax
import jax.numpy as jnp
from jax.experimental import pallas as pl


def kernel(feat, label, centers):
    raise NotImplementedError("write your pallas kernel here")


# speedup vs baseline: 1.9979x; 1.9979x over previous
import functools

import jax
import jax.numpy as jnp
from jax import lax
from jax.experimental import pallas as pl
from jax.experimental.pallas import tpu as pltpu
from jax.experimental.pallas import tpu_sc as plsc

BATCH = 16384
D = 128
LANES = 16
VECS = D // LANES

_info = plsc.get_sparse_core_info()
NC, NS = _info.num_cores, _info.num_subcores
NW = NC * NS
ROWS_W = BATCH // NW
CHUNK = 128
NCHUNK = ROWS_W // CHUNK


def _make_sc_call():
    mesh = plsc.VectorSubcoreMesh(core_axis_name="c", subcore_axis_name="s")

    @functools.partial(
        pl.kernel,
        mesh=mesh,
        out_type=jax.ShapeDtypeStruct((NW, LANES), jnp.float32),
        scratch_types=[
            pltpu.VMEM((NCHUNK, CHUNK), jnp.int32),
            pltpu.VMEM((2, CHUNK, D), jnp.float32),
            pltpu.VMEM((2, CHUNK, D), jnp.float32),
            pltpu.VMEM((LANES,), jnp.float32),
            pltpu.SemaphoreType.DMA,
            pltpu.SemaphoreType.DMA,
            pltpu.SemaphoreType.DMA,
            pltpu.SemaphoreType.DMA,
        ],
    )
    def sc_center_loss(feat_hbm, label_hbm, centers_hbm, out_hbm,
                       idx_v, cent_v, feat_v, out_v,
                       gsem0, gsem1, fsem0, fsem1):
        wid = lax.axis_index("s") * NC + lax.axis_index("c")
        base = wid * ROWS_W
        gsems = (gsem0, gsem1)
        fsems = (fsem0, fsem1)

        for c in range(NCHUNK):
            pltpu.sync_copy(label_hbm.at[pl.ds(base + c * CHUNK, CHUNK)],
                            idx_v.at[c])

        def start(c, slot):
            g = pltpu.async_copy(centers_hbm.at[idx_v.at[c]],
                                 cent_v.at[slot], gsems[slot])
            f = pltpu.async_copy(feat_hbm.at[pl.ds(base + c * CHUNK, CHUNK)],
                                 feat_v.at[slot], fsems[slot])
            return g, f

        def compute(slot, accs):
            fv = feat_v.at[slot]
            cv = cent_v.at[slot]

            def body(i, accs):
                new = []
                for j in range(VECS):
                    f = fv[i, pl.ds(j * LANES, LANES)]
                    cc = cv[i, pl.ds(j * LANES, LANES)]
                    d = f - cc
                    new.append(accs[j] + d * d)
                return tuple(new)

            return lax.fori_loop(0, CHUNK, body, accs)

        accs = tuple(jnp.zeros((LANES,), jnp.float32) for _ in range(VECS))
        copies = {0: start(0, 0)}
        for c in range(NCHUNK):
            if c + 1 < NCHUNK:
                copies[c + 1] = start(c + 1, (c + 1) % 2)
            g, f = copies.pop(c)
            g.wait()
            f.wait()
            accs = compute(c % 2, accs)

        total = accs[0]
        for j in range(1, VECS):
            total = total + accs[j]
        out_v[...] = total * 0.5
        pltpu.sync_copy(out_v, out_hbm.at[wid])

    return sc_center_loss


_sc_center_loss = _make_sc_call()


def kernel(feat, label, centers):
    partials = _sc_center_loss(feat, label.astype(jnp.int32), centers)
    return jnp.sum(partials)


# speedup vs baseline: 2.0751x; 1.0386x over previous
import functools

import jax
import jax.numpy as jnp
from jax import lax
from jax.experimental import pallas as pl
from jax.experimental.pallas import tpu as pltpu
from jax.experimental.pallas import tpu_sc as plsc

BATCH = 16384
D = 128
LANES = 16
VECS = D // LANES

_info = plsc.get_sparse_core_info()
NC, NS = _info.num_cores, _info.num_subcores
NW = NC * NS
ROWS_W = BATCH // NW
CHUNK = 128
NCHUNK = ROWS_W // CHUNK


def _make_sc_call():
    mesh = plsc.VectorSubcoreMesh(core_axis_name="c", subcore_axis_name="s")

    @functools.partial(
        pl.kernel,
        mesh=mesh,
        out_type=jax.ShapeDtypeStruct((NW, LANES), jnp.float32),
        scratch_types=[
            pltpu.VMEM((ROWS_W,), jnp.int32),
            pltpu.VMEM((2, CHUNK, D), jnp.float32),
            pltpu.VMEM((2, CHUNK, D), jnp.float32),
            pltpu.VMEM((LANES,), jnp.float32),
            pltpu.SemaphoreType.DMA,
            pltpu.SemaphoreType.DMA,
            pltpu.SemaphoreType.DMA,
            pltpu.SemaphoreType.DMA,
        ],
    )
    def sc_center_loss(feat_hbm, label_hbm, centers_hbm, out_hbm,
                       idx_v, cent_v, feat_v, out_v,
                       gsem0, gsem1, fsem0, fsem1):
        wid = lax.axis_index("s") * NC + lax.axis_index("c")
        base = wid * ROWS_W
        gsems = (gsem0, gsem1)
        fsems = (fsem0, fsem1)

        pltpu.sync_copy(label_hbm.at[pl.ds(base, ROWS_W)], idx_v)

        def start(c, slot):
            g = pltpu.async_copy(centers_hbm.at[idx_v.at[pl.ds(c * CHUNK, CHUNK)]],
                                 cent_v.at[slot], gsems[slot])
            f = pltpu.async_copy(feat_hbm.at[pl.ds(base + c * CHUNK, CHUNK)],
                                 feat_v.at[slot], fsems[slot])
            return g, f

        def compute(slot, accs):
            fv = feat_v.at[slot]
            cv = cent_v.at[slot]

            def body(i, accs):
                new = []
                for j in range(VECS):
                    f = fv[i, pl.ds(j * LANES, LANES)]
                    cc = cv[i, pl.ds(j * LANES, LANES)]
                    d = f - cc
                    new.append(accs[j] + d * d)
                return tuple(new)

            return lax.fori_loop(0, CHUNK, body, accs)

        accs = tuple(jnp.zeros((LANES,), jnp.float32) for _ in range(VECS))
        copies = {0: start(0, 0)}
        for c in range(NCHUNK):
            if c + 1 < NCHUNK:
                copies[c + 1] = start(c + 1, (c + 1) % 2)
            g, f = copies.pop(c)
            g.wait()
            f.wait()
            accs = compute(c % 2, accs)

        total = accs[0]
        for j in range(1, VECS):
            total = total + accs[j]
        out_v[...] = total * 0.5
        pltpu.sync_copy(out_v, out_hbm.at[wid])

    return sc_center_loss


_sc_center_loss = _make_sc_call()


def kernel(feat, label, centers):
    partials = _sc_center_loss(feat, label.astype(jnp.int32), centers)
    return jnp.sum(partials)
